# Initial kernel scaffold; baseline (speedup 1.0000x reference)
import jax
import jax.numpy as jnp
from jax.experimental import pallas as pl


def kernel(token_embedding, ctx, last_clip_labels, prompt_prefix_tokens, label_tokens):
    raise NotImplementedError("write your pallas kernel here")


# speedup vs baseline: 1.9989x; 1.9989x over previous
import functools

import jax
import jax.numpy as jnp
from jax import lax
from jax.experimental import pallas as pl
from jax.experimental.pallas import tpu as pltpu
from jax.experimental.pallas import tpu_sc as plsc

VOCAB = 49408
D = 512
N_CLS = 48
N_CTX = 8
MAX_LEN = 40
SAMPLE_RATE = 4
B = 8
T = 1024 // SAMPLE_RATE
P = 15
L_LAB = 5
SOS_ID = VOCAB - 2
EOS_ID = VOCAB - 1

NPAIR = B * T
NW = 32
PAIRS_PER_W = NPAIR // NW
BUF_ROWS = 48


def _sc_body(tok_emb, ctx_flat, tok16, labels, metab, metac, zrows,
             out, mask, buf, mbuf, pidx, labv, mb2, mc2, sem):
    wid = lax.axis_index("s") * 2 + lax.axis_index("c")
    wbase = wid * PAIRS_PER_W

    pltpu.sync_copy(tok16.at[pl.ds(wbase, PAIRS_PER_W)], pidx)
    pltpu.sync_copy(labels.at[pl.ds(wbase, PAIRS_PER_W)], labv)
    pltpu.async_copy(metab.at[labv], mb2, sem).wait()
    pltpu.async_copy(metac.at[labv], mc2, sem).wait()
    pltpu.sync_copy(zrows, buf.at[pl.ds(32, 16)])

    lanes = lax.iota(jnp.int32, 16)
    zero16 = jnp.zeros((16,), jnp.float32)
    mbuf[pl.ds(32, 16)] = zero16

    def body(p, carry):
        n = wbase + p
        pltpu.async_copy(tok_emb.at[pidx.at[p]], buf.at[pl.ds(0, 16)], sem).wait()
        pltpu.async_copy(ctx_flat.at[mb2.at[p, pl.ds(0, 8)]],
                         buf.at[pl.ds(16, 8)], sem).wait()
        pltpu.async_copy(tok_emb.at[mc2.at[p, pl.ds(0, 8)]],
                         buf.at[pl.ds(24, 8)], sem).wait()
        for r in (30, 31):
            for c in range(D // 16):
                buf[r, pl.ds(c * 16, 16)] = zero16
        acc0 = zero16
        for r in range(16):
            x = buf[r, pl.ds(0, 16)]
            acc0 = jnp.where(lanes == r, jnp.broadcast_to(x[0:1], (16,)), acc0)
        acc1 = zero16
        for r in range(16, 30):
            x = buf[r, pl.ds(0, 16)]
            acc1 = jnp.where(lanes == r - 16, jnp.broadcast_to(x[0:1], (16,)), acc1)
        one16 = jnp.ones((16,), jnp.float32)
        mbuf[pl.ds(0, 16)] = jnp.where(acc0 != 0.0, one16, zero16)
        mbuf[pl.ds(16, 16)] = jnp.where(acc1 != 0.0, one16, zero16)
        pltpu.sync_copy(buf.at[pl.ds(0, MAX_LEN)],
                        out.at[pl.ds(n * MAX_LEN, MAX_LEN)])
        pltpu.sync_copy(mbuf.at[pl.ds(0, MAX_LEN)],
                        mask.at[pl.ds(n * MAX_LEN, MAX_LEN)])
        return carry

    lax.fori_loop(0, PAIRS_PER_W, body, 0)


def kernel(token_embedding, ctx, last_clip_labels, prompt_prefix_tokens, label_tokens):
    labels_s = last_clip_labels[:, ::SAMPLE_RATE].reshape(NPAIR).astype(jnp.int32)
    tok16 = jnp.concatenate(
        [jnp.full((NPAIR, 1), SOS_ID, jnp.int32),
         prompt_prefix_tokens.reshape(NPAIR, P).astype(jnp.int32)], axis=1)
    ctx_flat = ctx.reshape(N_CLS * N_CTX, D)
    metab = jnp.pad(
        jnp.arange(N_CLS, dtype=jnp.int32)[:, None] * N_CTX
        + jnp.arange(N_CTX, dtype=jnp.int32)[None, :], ((0, 0), (0, 120)))
    metac = jnp.concatenate(
        [label_tokens.astype(jnp.int32),
         jnp.full((N_CLS, 128 - L_LAB), EOS_ID, jnp.int32)], axis=1)
    zrows = jnp.zeros((16, D), jnp.float32)

    mesh = plsc.VectorSubcoreMesh(core_axis_name="c", subcore_axis_name="s")
    run = functools.partial(
        pl.kernel,
        out_type=(jax.ShapeDtypeStruct((NPAIR * MAX_LEN, D), jnp.float32),
                  jax.ShapeDtypeStruct((NPAIR * MAX_LEN,), jnp.float32)),
        mesh=mesh,
        scratch_types=[
            pltpu.VMEM((BUF_ROWS, D), jnp.float32),
            pltpu.VMEM((BUF_ROWS,), jnp.float32),
            pltpu.VMEM((PAIRS_PER_W, 16), jnp.int32),
            pltpu.VMEM((PAIRS_PER_W,), jnp.int32),
            pltpu.VMEM((PAIRS_PER_W, 128), jnp.int32),
            pltpu.VMEM((PAIRS_PER_W, 128), jnp.int32),
            pltpu.SemaphoreType.DMA,
        ],
    )(_sc_body)
    out, mask = run(token_embedding, ctx_flat, tok16, labels_s,
                    metab, metac, zrows)
    prompts = out.reshape(B, T, MAX_LEN, D)
    pad_masks = mask.reshape(B, T, MAX_LEN, 1)
    return (prompts, pad_masks)


# speedup vs baseline: 2.0526x; 1.0268x over previous
import functools

import jax
import jax.numpy as jnp
from jax import lax
from jax.experimental import pallas as pl
from jax.experimental.pallas import tpu as pltpu
from jax.experimental.pallas import tpu_sc as plsc

VOCAB = 49408
D = 512
N_CLS = 48
N_CTX = 8
MAX_LEN = 40
SAMPLE_RATE = 4
B = 8
T = 1024 // SAMPLE_RATE
P = 15
L_LAB = 5
SOS_ID = VOCAB - 2
EOS_ID = VOCAB - 1

NPAIR = B * T
NW = 32
PAIRS_PER_W = NPAIR // NW
NBUF = 4
NBODY = PAIRS_PER_W // NBUF


def _sc_body(tok_emb, ctx_flat, tok16, labels, metab, metac, zrows,
             out, mask,
             b0, b1, b2, b3, m0, m1, m2, m3, pidx, labv, mb2, mc2,
             g0, g1, g2, g3, s0, s1, s2, s3):
    bufs = (b0, b1, b2, b3)
    mbufs = (m0, m1, m2, m3)
    gs = (g0, g1, g2, g3)
    ss = (s0, s1, s2, s3)
    wid = lax.axis_index("s") * 2 + lax.axis_index("c")
    wbase = wid * PAIRS_PER_W

    pltpu.sync_copy(tok16.at[pl.ds(wbase, PAIRS_PER_W)], pidx)
    pltpu.sync_copy(labels.at[pl.ds(wbase, PAIRS_PER_W)], labv)
    pltpu.async_copy(metab.at[labv], mb2, g0).wait()
    pltpu.async_copy(metac.at[labv], mc2, g0).wait()

    lanes = lax.iota(jnp.int32, 16)
    zero16 = jnp.zeros((16,), jnp.float32)
    one16 = jnp.ones((16,), jnp.float32)
    for b in range(NBUF):
        pltpu.sync_copy(zrows, bufs[b].at[pl.ds(32, 8)])
        mbufs[b][pl.ds(32, 16)] = zero16

    def body(i, carry):
        @pl.when(i > 0)
        def _():
            for b in range(NBUF):
                pltpu.make_async_copy(bufs[b].at[pl.ds(0, MAX_LEN)],
                                      out.at[pl.ds(0, MAX_LEN)], ss[b]).wait()
                pltpu.make_async_copy(mbufs[b].at[pl.ds(0, MAX_LEN)],
                                      mask.at[pl.ds(0, MAX_LEN)], ss[b]).wait()

        gds = []
        for b in range(NBUF):
            p = i * NBUF + b
            gds.append((
                pltpu.async_copy(tok_emb.at[pidx.at[p]],
                                 bufs[b].at[pl.ds(0, 16)], gs[b]),
                pltpu.async_copy(ctx_flat.at[mb2.at[p, pl.ds(0, 8)]],
                                 bufs[b].at[pl.ds(16, 8)], gs[b]),
                pltpu.async_copy(tok_emb.at[mc2.at[p, pl.ds(0, 8)]],
                                 bufs[b].at[pl.ds(24, 8)], gs[b]),
            ))

        for b in range(NBUF):
            for d in gds[b]:
                d.wait()
            buf, mbuf = bufs[b], mbufs[b]
            for r in (30, 31):
                for c in range(D // 16):
                    buf[r, pl.ds(c * 16, 16)] = zero16
            acc0 = zero16
            for r in range(16):
                x = buf[r, pl.ds(0, 16)]
                acc0 = jnp.where(lanes == r,
                                 jnp.broadcast_to(x[0:1], (16,)), acc0)
            acc1 = zero16
            for r in range(16, 30):
                x = buf[r, pl.ds(0, 16)]
                acc1 = jnp.where(lanes == r - 16,
                                 jnp.broadcast_to(x[0:1], (16,)), acc1)
            mbuf[pl.ds(0, 16)] = jnp.where(acc0 != 0.0, one16, zero16)
            mbuf[pl.ds(16, 16)] = jnp.where(acc1 != 0.0, one16, zero16)
            n = wbase + i * NBUF + b
            pltpu.async_copy(buf.at[pl.ds(0, MAX_LEN)],
                             out.at[pl.ds(n * MAX_LEN, MAX_LEN)], ss[b])
            pltpu.async_copy(mbuf.at[pl.ds(0, MAX_LEN)],
                             mask.at[pl.ds(n * MAX_LEN, MAX_LEN)], ss[b])
        return carry

    lax.fori_loop(0, NBODY, body, 0)

    for b in range(NBUF):
        pltpu.make_async_copy(bufs[b].at[pl.ds(0, MAX_LEN)],
                              out.at[pl.ds(0, MAX_LEN)], ss[b]).wait()
        pltpu.make_async_copy(mbufs[b].at[pl.ds(0, MAX_LEN)],
                              mask.at[pl.ds(0, MAX_LEN)], ss[b]).wait()


def kernel(token_embedding, ctx, last_clip_labels, prompt_prefix_tokens, label_tokens):
    labels_s = last_clip_labels[:, ::SAMPLE_RATE].reshape(NPAIR).astype(jnp.int32)
    tok16 = jnp.concatenate(
        [jnp.full((NPAIR, 1), SOS_ID, jnp.int32),
         prompt_prefix_tokens.reshape(NPAIR, P).astype(jnp.int32)], axis=1)
    ctx_flat = ctx.reshape(N_CLS * N_CTX, D)
    metab = jnp.pad(
        jnp.arange(N_CLS, dtype=jnp.int32)[:, None] * N_CTX
        + jnp.arange(N_CTX, dtype=jnp.int32)[None, :], ((0, 0), (0, 120)))
    metac = jnp.concatenate(
        [label_tokens.astype(jnp.int32),
         jnp.full((N_CLS, 128 - L_LAB), EOS_ID, jnp.int32)], axis=1)
    zrows = jnp.zeros((8, D), jnp.float32)

    mesh = plsc.VectorSubcoreMesh(core_axis_name="c", subcore_axis_name="s")
    run = functools.partial(
        pl.kernel,
        out_type=(jax.ShapeDtypeStruct((NPAIR * MAX_LEN, D), jnp.float32),
                  jax.ShapeDtypeStruct((NPAIR * MAX_LEN,), jnp.float32)),
        mesh=mesh,
        scratch_types=(
            [pltpu.VMEM((MAX_LEN, D), jnp.float32)] * NBUF
            + [pltpu.VMEM((48,), jnp.float32)] * NBUF
            + [pltpu.VMEM((PAIRS_PER_W, 16), jnp.int32),
               pltpu.VMEM((PAIRS_PER_W,), jnp.int32),
               pltpu.VMEM((PAIRS_PER_W, 128), jnp.int32),
               pltpu.VMEM((PAIRS_PER_W, 128), jnp.int32)]
            + [pltpu.SemaphoreType.DMA] * (2 * NBUF)
        ),
    )(_sc_body)
    out, mask = run(token_embedding, ctx_flat, tok16, labels_s,
                    metab, metac, zrows)
    prompts = out.reshape(B, T, MAX_LEN, D)
    pad_masks = mask.reshape(B, T, MAX_LEN, 1)
    return (prompts, pad_masks)


# speedup vs baseline: 2.0535x; 1.0005x over previous
import functools

import jax
import jax.numpy as jnp
from jax import lax
from jax.experimental import pallas as pl
from jax.experimental.pallas import tpu as pltpu
from jax.experimental.pallas import tpu_sc as plsc

VOCAB = 49408
D = 512
N_CLS = 48
N_CTX = 8
MAX_LEN = 40
SAMPLE_RATE = 4
B = 8
T = 1024 // SAMPLE_RATE
P = 15
L_LAB = 5
SOS_ID = VOCAB - 2
EOS_ID = VOCAB - 1

NPAIR = B * T
NW = 32
PAIRS_PER_W = NPAIR // NW
NBUF = 4
NBODY = PAIRS_PER_W // NBUF


def _sc_body(tok_emb, ctx_flat, tok16, labels, metab, metac, zrows,
             out, mask,
             b0, b1, b2, b3, mmall, pidx, labv, mb2, mc2,
             g0, g1, g2, g3, s0, s1, s2, s3):
    bufs = (b0, b1, b2, b3)
    gs = (g0, g1, g2, g3)
    ss = (s0, s1, s2, s3)
    wid = lax.axis_index("s") * 2 + lax.axis_index("c")
    wbase = wid * PAIRS_PER_W

    pltpu.sync_copy(tok16.at[pl.ds(wbase, PAIRS_PER_W)], pidx)
    pltpu.sync_copy(labels.at[pl.ds(wbase, PAIRS_PER_W)], labv)
    pltpu.async_copy(metab.at[labv], mb2, g0).wait()
    pltpu.async_copy(metac.at[labv], mc2, g0).wait()

    lanes = lax.iota(jnp.int32, 16)
    zero16 = jnp.zeros((16,), jnp.float32)
    one16 = jnp.ones((16,), jnp.float32)
    for b in range(NBUF):
        pltpu.sync_copy(zrows, bufs[b].at[pl.ds(32, 8)])

    def body(i, carry):
        @pl.when(i > 0)
        def _():
            for b in range(NBUF):
                pltpu.make_async_copy(bufs[b].at[pl.ds(0, MAX_LEN)],
                                      out.at[pl.ds(0, MAX_LEN)], ss[b]).wait()

        gds = []
        for b in range(NBUF):
            p = i * NBUF + b
            gds.append((
                pltpu.async_copy(tok_emb.at[pidx.at[p]],
                                 bufs[b].at[pl.ds(0, 16)], gs[b]),
                pltpu.async_copy(ctx_flat.at[mb2.at[p, pl.ds(0, 8)]],
                                 bufs[b].at[pl.ds(16, 8)], gs[b]),
                pltpu.async_copy(tok_emb.at[mc2.at[p, pl.ds(0, 8)]],
                                 bufs[b].at[pl.ds(24, 8)], gs[b]),
            ))

        for b in range(NBUF):
            for d in gds[b]:
                d.wait()
            buf = bufs[b]
            for r in (30, 31):
                for c in range(D // 16):
                    buf[r, pl.ds(c * 16, 16)] = zero16
            acc0 = zero16
            for r in range(16):
                x = buf[r, pl.ds(0, 16)]
                acc0 = jnp.where(lanes == r,
                                 jnp.broadcast_to(x[0:1], (16,)), acc0)
            acc1 = zero16
            for r in range(16, 30):
                x = buf[r, pl.ds(0, 16)]
                acc1 = jnp.where(lanes == r - 16,
                                 jnp.broadcast_to(x[0:1], (16,)), acc1)
            poff = (i * NBUF + b) * MAX_LEN
            mmall[pl.ds(poff, 16)] = jnp.where(acc0 != 0.0, one16, zero16)
            mmall[pl.ds(poff + 16, 16)] = jnp.where(acc1 != 0.0, one16, zero16)
            mmall[pl.ds(poff + 32, 16)] = zero16
            n = wbase + i * NBUF + b
            pltpu.async_copy(buf.at[pl.ds(0, MAX_LEN)],
                             out.at[pl.ds(n * MAX_LEN, MAX_LEN)], ss[b])
        return carry

    lax.fori_loop(0, NBODY, body, 0)

    for b in range(NBUF):
        pltpu.make_async_copy(bufs[b].at[pl.ds(0, MAX_LEN)],
                              out.at[pl.ds(0, MAX_LEN)], ss[b]).wait()
    pltpu.sync_copy(mmall.at[pl.ds(0, PAIRS_PER_W * MAX_LEN)],
                    mask.at[pl.ds(wbase * MAX_LEN, PAIRS_PER_W * MAX_LEN)])


def kernel(token_embedding, ctx, last_clip_labels, prompt_prefix_tokens, label_tokens):
    labels_s = last_clip_labels[:, ::SAMPLE_RATE].reshape(NPAIR).astype(jnp.int32)
    tok16 = jnp.concatenate(
        [jnp.full((NPAIR, 1), SOS_ID, jnp.int32),
         prompt_prefix_tokens.reshape(NPAIR, P).astype(jnp.int32)], axis=1)
    ctx_flat = ctx.reshape(N_CLS * N_CTX, D)
    metab = jnp.pad(
        jnp.arange(N_CLS, dtype=jnp.int32)[:, None] * N_CTX
        + jnp.arange(N_CTX, dtype=jnp.int32)[None, :], ((0, 0), (0, 120)))
    metac = jnp.concatenate(
        [label_tokens.astype(jnp.int32),
         jnp.full((N_CLS, 128 - L_LAB), EOS_ID, jnp.int32)], axis=1)
    zrows = jnp.zeros((8, D), jnp.float32)

    mesh = plsc.VectorSubcoreMesh(core_axis_name="c", subcore_axis_name="s")
    run = functools.partial(
        pl.kernel,
        out_type=(jax.ShapeDtypeStruct((NPAIR * MAX_LEN, D), jnp.float32),
                  jax.ShapeDtypeStruct((NPAIR * MAX_LEN,), jnp.float32)),
        mesh=mesh,
        scratch_types=(
            [pltpu.VMEM((MAX_LEN, D), jnp.float32)] * NBUF
            + [pltpu.VMEM((PAIRS_PER_W * MAX_LEN + 8,), jnp.float32)]
            + [pltpu.VMEM((PAIRS_PER_W, 16), jnp.int32),
               pltpu.VMEM((PAIRS_PER_W,), jnp.int32),
               pltpu.VMEM((PAIRS_PER_W, 128), jnp.int32),
               pltpu.VMEM((PAIRS_PER_W, 128), jnp.int32)]
            + [pltpu.SemaphoreType.DMA] * (2 * NBUF)
        ),
    )(_sc_body)
    out, mask = run(token_embedding, ctx_flat, tok16, labels_s,
                    metab, metac, zrows)
    prompts = out.reshape(B, T, MAX_LEN, D)
    pad_masks = mask.reshape(B, T, MAX_LEN, 1)
    return (prompts, pad_masks)
